# Initial kernel scaffold; baseline (speedup 1.0000x reference)
import jax
import jax.numpy as jnp
from jax.experimental import pallas as pl


def kernel(x, emb):
    raise NotImplementedError("write your pallas kernel here")


# speedup vs baseline: 1.8423x; 1.8423x over previous
import functools

import jax
import jax.numpy as jnp
from jax import lax
from jax.experimental import pallas as pl
from jax.experimental.pallas import tpu as pltpu
from jax.experimental.pallas import tpu_sc as plsc

D = 64
IPS = 128
K = 5
C = K * IPS


@functools.lru_cache(maxsize=None)
def _build(n_idx: int, vocab: int):
    info = plsc.get_sparse_core_info()
    nc, ns = info.num_cores, info.num_subcores
    nw = nc * ns
    assert n_idx % (nw * C) == 0
    nb = n_idx // nw
    n_rows = nb // IPS
    n_groups = nb // C

    mesh = plsc.VectorSubcoreMesh(core_axis_name="c", subcore_axis_name="s")

    @functools.partial(
        pl.kernel,
        out_type=jax.ShapeDtypeStruct((n_idx, D), jnp.float32),
        mesh=mesh,
        scratch_types=[
            pltpu.VMEM((n_rows, IPS), jnp.int32),
            pltpu.VMEM((C, D), jnp.float32),
            pltpu.SemaphoreType.DMA,
        ],
        compiler_params=pltpu.CompilerParams(use_tc_tiling_on_sc=False),
    )
    def emb_kernel(x_hbm, emb_hbm, out_hbm, idx_v, rows_v, gsem):
        wid = lax.axis_index("s") * nc + lax.axis_index("c")
        pltpu.sync_copy(x_hbm.at[pl.ds(wid * n_rows, n_rows)], idx_v)
        out_base = wid * nb

        def g_body(g, carry):
            descs = []
            for j in range(K):
                d = pltpu.make_async_copy(
                    emb_hbm.at[idx_v.at[g * K + j]],
                    rows_v.at[pl.ds(j * IPS, IPS)],
                    gsem,
                )
                d.start()
                descs.append(d)
            for d in descs:
                d.wait()
            pltpu.sync_copy(rows_v, out_hbm.at[pl.ds(out_base + g * C, C)])
            return carry

        lax.fori_loop(0, n_groups, g_body, 0)

    return emb_kernel


def kernel(x, emb):
    b, l = x.shape
    n_idx = b * l
    x_flat = x.astype(jnp.int32).reshape(n_idx // IPS, IPS)
    out = _build(n_idx, emb.shape[0])(x_flat, emb)
    return out.reshape(b, l, D)


# speedup vs baseline: 1.8756x; 1.0180x over previous
import functools

import jax
import jax.numpy as jnp
from jax import lax
from jax.experimental import pallas as pl
from jax.experimental.pallas import tpu as pltpu
from jax.experimental.pallas import tpu_sc as plsc

D = 64
IPS = 128
K = 5
C = K * IPS


@functools.lru_cache(maxsize=None)
def _build(n_idx: int, vocab: int):
    info = plsc.get_sparse_core_info()
    nc, ns = info.num_cores, info.num_subcores
    nw = nc * ns
    assert n_idx % (nw * C) == 0
    nb = n_idx // nw
    n_rows = nb // IPS
    n_groups = nb // C

    mesh = plsc.VectorSubcoreMesh(core_axis_name="c", subcore_axis_name="s")

    @functools.partial(
        pl.kernel,
        out_type=jax.ShapeDtypeStruct((n_idx, D), jnp.float32),
        mesh=mesh,
        scratch_types=[
            pltpu.VMEM((n_rows, IPS), jnp.int32),
            pltpu.VMEM((C, D), jnp.float32),
            pltpu.VMEM((C, D), jnp.float32),
            pltpu.SemaphoreType.DMA,
            pltpu.SemaphoreType.DMA,
        ],
        compiler_params=pltpu.CompilerParams(use_tc_tiling_on_sc=False),
    )
    def emb_kernel(x_hbm, emb_hbm, out_hbm, idx_v, rows_a, rows_b, sem_a,
                   sem_b):
        wid = lax.axis_index("s") * nc + lax.axis_index("c")
        pltpu.sync_copy(x_hbm.at[pl.ds(wid * n_rows, n_rows)], idx_v)
        out_base = wid * nb
        bufs = ((rows_a, sem_a), (rows_b, sem_b))

        def issue(g, rows, sem):
            for j in range(K):
                pltpu.make_async_copy(
                    emb_hbm.at[idx_v.at[g * K + j]],
                    rows.at[pl.ds(j * IPS, IPS)],
                    sem,
                ).start()

        def drain(g, rows, sem):
            for j in range(K):
                pltpu.make_async_copy(
                    emb_hbm.at[idx_v.at[g * K + j]],
                    rows.at[pl.ds(j * IPS, IPS)],
                    sem,
                ).wait()

        issue(0, rows_a, sem_a)
        issue(1, rows_b, sem_b)

        def pair_body(p, carry):
            g = 2 * p
            for parity, (rows, sem) in enumerate(bufs):
                gg = g + parity
                drain(gg, rows, sem)
                pltpu.sync_copy(rows,
                                out_hbm.at[pl.ds(out_base + gg * C, C)])

                @pl.when(gg + 2 < n_groups)
                def _():
                    issue(gg + 2, rows, sem)

            return carry

        lax.fori_loop(0, n_groups // 2, pair_body, 0)

    return emb_kernel


def kernel(x, emb):
    b, l = x.shape
    n_idx = b * l
    x_flat = x.astype(jnp.int32).reshape(n_idx // IPS, IPS)
    out = _build(n_idx, emb.shape[0])(x_flat, emb)
    return out.reshape(b, l, D)


# speedup vs baseline: 1.8875x; 1.0063x over previous
import functools

import jax
import jax.numpy as jnp
from jax import lax
from jax.experimental import pallas as pl
from jax.experimental.pallas import tpu as pltpu
from jax.experimental.pallas import tpu_sc as plsc

D = 64
IPS = 128
K = 5
C = K * IPS


@functools.lru_cache(maxsize=None)
def _build(n_idx: int, vocab: int):
    info = plsc.get_sparse_core_info()
    nc, ns = info.num_cores, info.num_subcores
    nw = nc * ns
    assert n_idx % (nw * C) == 0
    nb = n_idx // nw
    n_rows = nb // IPS
    n_groups = nb // C

    mesh = plsc.VectorSubcoreMesh(core_axis_name="c", subcore_axis_name="s")

    @functools.partial(
        pl.kernel,
        out_type=jax.ShapeDtypeStruct((n_idx, D), jnp.float32),
        mesh=mesh,
        scratch_types=[
            pltpu.VMEM((nb,), jnp.int32),
            pltpu.VMEM((C, D), jnp.float32),
            pltpu.VMEM((C, D), jnp.float32),
            pltpu.SemaphoreType.DMA,
            pltpu.SemaphoreType.DMA,
        ],
        compiler_params=pltpu.CompilerParams(use_tc_tiling_on_sc=False),
    )
    def emb_kernel(x_hbm, emb_hbm, out_hbm, idx_v, rows_a, rows_b, sem_a,
                   sem_b):
        wid = lax.axis_index("s") * nc + lax.axis_index("c")
        pltpu.sync_copy(x_hbm.at[pl.ds(wid * nb, nb)], idx_v)
        out_base = wid * nb
        bufs = ((rows_a, sem_a), (rows_b, sem_b))

        def issue(g, rows, sem):
            pltpu.make_async_copy(
                emb_hbm.at[idx_v.at[pl.ds(g * C, C)]], rows, sem,
            ).start()

        def drain(g, rows, sem):
            pltpu.make_async_copy(
                emb_hbm.at[idx_v.at[pl.ds(g * C, C)]], rows, sem,
            ).wait()

        issue(0, rows_a, sem_a)
        issue(1, rows_b, sem_b)

        def pair_body(p, carry):
            g = 2 * p
            for parity, (rows, sem) in enumerate(bufs):
                gg = g + parity
                drain(gg, rows, sem)
                pltpu.sync_copy(rows,
                                out_hbm.at[pl.ds(out_base + gg * C, C)])

                @pl.when(gg + 2 < n_groups)
                def _():
                    issue(gg + 2, rows, sem)

            return carry

        lax.fori_loop(0, n_groups // 2, pair_body, 0)

    return emb_kernel


def kernel(x, emb):
    b, l = x.shape
    n_idx = b * l
    x_flat = x.astype(jnp.int32).reshape(n_idx)
    out = _build(n_idx, emb.shape[0])(x_flat, emb)
    return out.reshape(b, l, D)
